# baseline (speedup 1.0000x reference)
import jax
import jax.numpy as jnp
from jax.experimental import pallas as pl
from jax.experimental.pallas import tpu as pltpu

_BM = 512
_BKW = 1024
_NH = 1024
_NSTAGE = 4
_VMEM_LIMIT = 64 * 1024 * 1024


def _matmul_bias_kernel(x_ref, w_ref, b_ref, o_ref, wbf_ref):
    s = pl.program_id(1)

    @pl.when(s < _NSTAGE)
    def _():
        chunk = w_ref[...].astype(jnp.bfloat16)
        wbf_ref[:, pl.ds(s * _BKW, _BKW)] = chunk

        @pl.when(s == 0)
        def _():
            o_ref[...] = jnp.broadcast_to(b_ref[...], o_ref.shape)

        o_ref[...] += jax.lax.dot_general(
            x_ref[:, pl.ds(s * _BKW, _BKW)].astype(jnp.bfloat16),
            chunk,
            dimension_numbers=(((1,), (1,)), ((), ())),
            preferred_element_type=jnp.float32,
        )

    @pl.when(s >= _NSTAGE)
    def _():
        o_ref[...] = (
            jax.lax.dot_general(
                x_ref[...].astype(jnp.bfloat16),
                wbf_ref[...],
                dimension_numbers=(((1,), (1,)), ((), ())),
                preferred_element_type=jnp.float32,
            )
            + b_ref[...]
        )


@jax.jit
def kernel(x, weight, bias):
    B, D_in = x.shape
    D_out = weight.shape[0]

    b2 = bias.reshape(1, D_out)
    n_grid = D_out // _NH
    n_steps = _NSTAGE + B // _BM - 1

    return pl.pallas_call(
        _matmul_bias_kernel,
        out_shape=jax.ShapeDtypeStruct((B, D_out), jnp.float32),
        grid=(n_grid, n_steps),
        in_specs=[
            pl.BlockSpec(
                (_BM, D_in),
                lambda n, s: (jnp.where(s < _NSTAGE, 0, s - _NSTAGE + 1), 0),
            ),
            pl.BlockSpec(
                (_NH, _BKW),
                lambda n, s: (n, jnp.where(s < _NSTAGE, s, _NSTAGE - 1)),
            ),
            pl.BlockSpec((1, _NH), lambda n, s: (0, n)),
        ],
        out_specs=pl.BlockSpec(
            (_BM, _NH),
            lambda n, s: (jnp.where(s < _NSTAGE, 0, s - _NSTAGE + 1), n),
        ),
        scratch_shapes=[pltpu.VMEM((_NH, D_in), jnp.bfloat16)],
        compiler_params=pltpu.CompilerParams(
            dimension_semantics=("parallel", "arbitrary"),
            vmem_limit_bytes=_VMEM_LIMIT,
        ),
    )(x, weight, b2)


# speedup vs baseline: 1.0549x; 1.0549x over previous
import jax
import jax.numpy as jnp
from jax.experimental import pallas as pl
from jax.experimental.pallas import tpu as pltpu

_BM = 512
_BKW = 512
_NH = 2048
_NSTAGE = 8
_VMEM_LIMIT = 64 * 1024 * 1024


def _matmul_bias_kernel(x_ref, w_ref, b_ref, o_ref, wbf_ref):
    s = pl.program_id(1)

    @pl.when(s < _NSTAGE)
    def _():
        chunk = w_ref[...].astype(jnp.bfloat16)
        wbf_ref[:, pl.ds(s * _BKW, _BKW)] = chunk

        @pl.when(s == 0)
        def _():
            o_ref[...] = jnp.broadcast_to(b_ref[...], o_ref.shape)

        o_ref[...] += jax.lax.dot_general(
            x_ref[:, pl.ds(s * _BKW, _BKW)].astype(jnp.bfloat16),
            chunk,
            dimension_numbers=(((1,), (1,)), ((), ())),
            preferred_element_type=jnp.float32,
        )

    @pl.when(s >= _NSTAGE)
    def _():
        o_ref[...] = (
            jax.lax.dot_general(
                x_ref[...].astype(jnp.bfloat16),
                wbf_ref[...],
                dimension_numbers=(((1,), (1,)), ((), ())),
                preferred_element_type=jnp.float32,
            )
            + b_ref[...]
        )


@jax.jit
def kernel(x, weight, bias):
    B, D_in = x.shape
    D_out = weight.shape[0]

    b2 = bias.reshape(1, D_out)
    n_grid = D_out // _NH
    n_steps = _NSTAGE + B // _BM - 1

    return pl.pallas_call(
        _matmul_bias_kernel,
        out_shape=jax.ShapeDtypeStruct((B, D_out), jnp.float32),
        grid=(n_grid, n_steps),
        in_specs=[
            pl.BlockSpec(
                (_BM, D_in),
                lambda n, s: (jnp.where(s < _NSTAGE, 0, s - _NSTAGE + 1), 0),
            ),
            pl.BlockSpec(
                (_NH, _BKW),
                lambda n, s: (n, jnp.where(s < _NSTAGE, s, _NSTAGE - 1)),
            ),
            pl.BlockSpec((1, _NH), lambda n, s: (0, n)),
        ],
        out_specs=pl.BlockSpec(
            (_BM, _NH),
            lambda n, s: (jnp.where(s < _NSTAGE, 0, s - _NSTAGE + 1), n),
        ),
        scratch_shapes=[pltpu.VMEM((_NH, D_in), jnp.bfloat16)],
        compiler_params=pltpu.CompilerParams(
            dimension_semantics=("parallel", "arbitrary"),
            vmem_limit_bytes=_VMEM_LIMIT,
        ),
    )(x, weight, b2)


# speedup vs baseline: 1.1138x; 1.0559x over previous
import jax
import jax.numpy as jnp
from jax.experimental import pallas as pl
from jax.experimental.pallas import tpu as pltpu

_BM = 512
_BKW = 1024
_NH = 2048
_NSTAGE = 4
_VMEM_LIMIT = 64 * 1024 * 1024


def _matmul_bias_kernel(x_ref, w_ref, b_ref, o_ref, wbf_ref):
    s = pl.program_id(1)

    @pl.when(s < _NSTAGE)
    def _():
        chunk = w_ref[...].astype(jnp.bfloat16)
        wbf_ref[:, pl.ds(s * _BKW, _BKW)] = chunk

        @pl.when(s == 0)
        def _():
            o_ref[...] = jnp.broadcast_to(b_ref[...], o_ref.shape)

        o_ref[...] += jax.lax.dot_general(
            x_ref[:, pl.ds(s * _BKW, _BKW)].astype(jnp.bfloat16),
            chunk,
            dimension_numbers=(((1,), (1,)), ((), ())),
            preferred_element_type=jnp.float32,
        )

    @pl.when(s >= _NSTAGE)
    def _():
        o_ref[...] = (
            jax.lax.dot_general(
                x_ref[...].astype(jnp.bfloat16),
                wbf_ref[...],
                dimension_numbers=(((1,), (1,)), ((), ())),
                preferred_element_type=jnp.float32,
            )
            + b_ref[...]
        )


@jax.jit
def kernel(x, weight, bias):
    B, D_in = x.shape
    D_out = weight.shape[0]

    b2 = bias.reshape(1, D_out)
    n_grid = D_out // _NH
    n_steps = _NSTAGE + B // _BM - 1

    return pl.pallas_call(
        _matmul_bias_kernel,
        out_shape=jax.ShapeDtypeStruct((B, D_out), jnp.float32),
        grid=(n_grid, n_steps),
        in_specs=[
            pl.BlockSpec(
                (_BM, D_in),
                lambda n, s: (jnp.where(s < _NSTAGE, 0, s - _NSTAGE + 1), 0),
            ),
            pl.BlockSpec(
                (_NH, _BKW),
                lambda n, s: (n, jnp.where(s < _NSTAGE, s, _NSTAGE - 1)),
            ),
            pl.BlockSpec((1, _NH), lambda n, s: (0, n)),
        ],
        out_specs=pl.BlockSpec(
            (_BM, _NH),
            lambda n, s: (jnp.where(s < _NSTAGE, 0, s - _NSTAGE + 1), n),
        ),
        scratch_shapes=[pltpu.VMEM((_NH, D_in), jnp.bfloat16)],
        compiler_params=pltpu.CompilerParams(
            dimension_semantics=("parallel", "arbitrary"),
            vmem_limit_bytes=_VMEM_LIMIT,
        ),
    )(x, weight, b2)


# speedup vs baseline: 1.1214x; 1.0068x over previous
import jax
import jax.numpy as jnp
from jax.experimental import pallas as pl
from jax.experimental.pallas import tpu as pltpu

_BM = 512
_BKW = 1024
_NH = 2048
_NSTAGE = 4
_VMEM_LIMIT = 64 * 1024 * 1024


def _matmul_bias_kernel(x_ref, w_ref, b_ref, o_ref, wbf_ref):
    s = pl.program_id(1)

    @pl.when(s < _NSTAGE)
    def _():
        chunk = w_ref[...].astype(jnp.bfloat16)
        wbf_ref[:, pl.ds(s * _BKW, _BKW)] = chunk

        part = jax.lax.dot_general(
            x_ref[:, pl.ds(s * _BKW, _BKW)].astype(jnp.bfloat16),
            chunk,
            dimension_numbers=(((1,), (1,)), ((), ())),
            preferred_element_type=jnp.float32,
        )

        @pl.when(s == 0)
        def _():
            o_ref[...] = part + b_ref[...]

        @pl.when(s > 0)
        def _():
            o_ref[...] += part

    @pl.when(s >= _NSTAGE)
    def _():
        o_ref[...] = (
            jax.lax.dot_general(
                x_ref[...].astype(jnp.bfloat16),
                wbf_ref[...],
                dimension_numbers=(((1,), (1,)), ((), ())),
                preferred_element_type=jnp.float32,
            )
            + b_ref[...]
        )


@jax.jit
def kernel(x, weight, bias):
    B, D_in = x.shape
    D_out = weight.shape[0]

    b2 = bias.reshape(1, D_out)
    n_grid = D_out // _NH
    n_steps = _NSTAGE + B // _BM - 1

    return pl.pallas_call(
        _matmul_bias_kernel,
        out_shape=jax.ShapeDtypeStruct((B, D_out), jnp.float32),
        grid=(n_grid, n_steps),
        in_specs=[
            pl.BlockSpec(
                (_BM, D_in),
                lambda n, s: (jnp.where(s < _NSTAGE, 0, s - _NSTAGE + 1), 0),
            ),
            pl.BlockSpec(
                (_NH, _BKW),
                lambda n, s: (n, jnp.where(s < _NSTAGE, s, _NSTAGE - 1)),
            ),
            pl.BlockSpec((1, _NH), lambda n, s: (0, n)),
        ],
        out_specs=pl.BlockSpec(
            (_BM, _NH),
            lambda n, s: (jnp.where(s < _NSTAGE, 0, s - _NSTAGE + 1), n),
        ),
        scratch_shapes=[pltpu.VMEM((_NH, D_in), jnp.bfloat16)],
        compiler_params=pltpu.CompilerParams(
            dimension_semantics=("parallel", "arbitrary"),
            vmem_limit_bytes=_VMEM_LIMIT,
        ),
    )(x, weight, b2)
